# baseline (speedup 1.0000x reference)
import functools

import jax
import jax.numpy as jnp
from jax import lax
from jax.experimental import pallas as pl
from jax.experimental.pallas import tpu as pltpu
from jax.experimental.pallas import tpu_sc as plsc

S_LEN = 2048
D_MODEL = 1024
KV_R = 64
N_HD = 16
N_EXP = 16
M_INT = 512
S_INT = 2048
N_VOCAB = 32000
EPS = 1e-5
SBLK = 256
N_SBLK = S_INT // SBLK
EBLK = 256
E_STEPS = 2 * N_EXP
VBLK = 640
N_VBLK = N_VOCAB // VBLK
PHASE = 1 + E_STEPS + N_SBLK

SC_NC = 2
SC_NS = 16
SC_NW = SC_NC * SC_NS
BPW = S_LEN // SC_NW


def _ln_rows(x, w, b):
    mu = jnp.mean(x, axis=-1, keepdims=True)
    var = jnp.mean((x - mu) ** 2, axis=-1, keepdims=True)
    return (x - mu) / jnp.sqrt(var + EPS) * w + b


def _silu(x):
    return x * jax.nn.sigmoid(x)


def _dot_t(a, b):
    return lax.dot_general(a, b, (((1,), (1,)), ((), ())),
                           preferred_element_type=jnp.float32)


def _dot_t16(a, b):
    return lax.dot_general(a.astype(jnp.bfloat16), b.astype(jnp.bfloat16),
                           (((1,), (1,)), ((), ())),
                           preferred_element_type=jnp.float32)


def _embed_body(table_hbm, idx_hbm, out_hbm, idx_v, rows_v, sem):
    wid = lax.axis_index("s") * SC_NC + lax.axis_index("c")
    base = wid * BPW
    pltpu.sync_copy(idx_hbm.at[pl.ds(base, BPW)], idx_v)
    pltpu.async_copy(table_hbm.at[idx_v], rows_v, sem).wait()
    pltpu.sync_copy(rows_v, out_hbm.at[pl.ds(base, BPW)])


def _embed_gather(table, ids):
    mesh = plsc.VectorSubcoreMesh(core_axis_name="c", subcore_axis_name="s")
    k = functools.partial(
        pl.kernel,
        mesh=mesh,
        out_type=jax.ShapeDtypeStruct((S_LEN, D_MODEL), jnp.float32),
        scratch_types=[
            pltpu.VMEM((BPW,), jnp.int32),
            pltpu.VMEM((BPW, D_MODEL), jnp.float32),
            pltpu.SemaphoreType.DMA,
        ],
    )(_embed_body)
    return k(table, ids)


def _prologue(x, refs, x_out, xbf_s, g_s):
    (n1w, n1b, wv, wo, n2w, n2b, wr, br) = refs
    xln = _ln_rows(x, n1w[...], n1b[...])
    v = _dot_t(xln, wv[...])
    vt = jnp.concatenate([v] * N_HD, axis=1)
    y = x + _dot_t(vt, wo[...])
    x_out[...] = y
    xln2 = _ln_rows(y, n2w[...], n2b[...])
    xbf_s[...] = xln2.astype(jnp.bfloat16)
    logits = _dot_t(xln2, wr[...]) + br[...]
    m = jnp.max(logits, axis=-1, keepdims=True)
    ex = jnp.exp(logits - m)
    w = ex / jnp.sum(ex, axis=-1, keepdims=True)
    m1 = jnp.max(w, axis=-1, keepdims=True)
    w2 = jnp.where(w >= m1, -1.0, w)
    m2 = jnp.max(w2, axis=-1, keepdims=True)
    sel = w >= m2
    g_s[...] = jnp.where(sel, w, 0.0) / (m1 + m2)


def _expert_step(t, we1, be1, we2, be2, x_out, xbf_s, g_s):
    e = t // 2
    xb = xbf_s[...]
    h = _silu(_dot_t16(xb, we1[0]) + be1[0])
    o = _dot_t16(h, we2[0])
    first = jnp.where(t % 2 == 0, 1.0, 0.0)
    lane = lax.broadcasted_iota(jnp.int32, (S_LEN, N_EXP), 1)
    ge = jnp.sum(jnp.where(lane == e, g_s[...], 0.0), axis=1, keepdims=True)
    x_out[...] = x_out[...] + ge * (o + first * be2[0])


def _shared_step(c, ws1, bs1, ws2, bs2, x_out, xbf_s, snapshot):
    xb = xbf_s[...]
    h = _silu(_dot_t16(xb, ws1[...]) + bs1[...])
    part = _dot_t16(h, ws2[...])
    first = jnp.where(c == 0, 1.0, 0.0)
    newx = x_out[...] + part + first * bs2[...]
    x_out[...] = newx
    if snapshot:
        @pl.when(c == N_SBLK - 1)
        def _():
            xbf_s[...] = newx.astype(jnp.bfloat16)


def _layer_specs(base):
    def _t_idx(s):
        return jnp.clip(s - (base + 1), 0, E_STEPS - 1)

    def _e_idx(s):
        return _t_idx(s) // 2

    def _h_idx(s):
        return _t_idx(s) % 2

    def _c_idx(s):
        return jnp.clip(s - (base + 1 + E_STEPS), 0, N_SBLK - 1)

    return [
        pl.BlockSpec((1, D_MODEL), lambda s: (0, 0)),
        pl.BlockSpec((1, D_MODEL), lambda s: (0, 0)),
        pl.BlockSpec((KV_R, D_MODEL), lambda s: (0, 0)),
        pl.BlockSpec((D_MODEL, D_MODEL), lambda s: (0, 0)),
        pl.BlockSpec((1, D_MODEL), lambda s: (0, 0)),
        pl.BlockSpec((1, D_MODEL), lambda s: (0, 0)),
        pl.BlockSpec((N_EXP, D_MODEL), lambda s: (0, 0)),
        pl.BlockSpec((1, N_EXP), lambda s: (0, 0)),
        pl.BlockSpec((1, EBLK, D_MODEL),
                     lambda s: (_e_idx(s), _h_idx(s), 0)),
        pl.BlockSpec((1, 1, EBLK), lambda s: (_e_idx(s), 0, _h_idx(s))),
        pl.BlockSpec((1, D_MODEL, EBLK),
                     lambda s: (_e_idx(s), 0, _h_idx(s))),
        pl.BlockSpec((1, 1, D_MODEL), lambda s: (_e_idx(s), 0, 0)),
        pl.BlockSpec((SBLK, D_MODEL), lambda s: (_c_idx(s), 0)),
        pl.BlockSpec((1, SBLK), lambda s: (0, _c_idx(s))),
        pl.BlockSpec((D_MODEL, SBLK), lambda s: (0, _c_idx(s))),
        pl.BlockSpec((1, D_MODEL), lambda s: (0, 0)),
    ]


def _layer_args(lp):
    return (
        lp['n1w'].reshape(1, D_MODEL), lp['n1b'].reshape(1, D_MODEL),
        lp['Wv'], lp['Wo'],
        lp['n2w'].reshape(1, D_MODEL), lp['n2b'].reshape(1, D_MODEL),
        lp['Wr'], lp['br'].reshape(1, N_EXP),
        lp['We1'], lp['be1'].reshape(N_EXP, 1, M_INT),
        lp['We2'], lp['be2'].reshape(N_EXP, 1, D_MODEL),
        lp['Ws1'][0], lp['bs1'], lp['Ws2'][0], lp['bs2'],
    )


def _l1_body(x_ref,
             n1w, n1b, wv, wo, n2w, n2b, wr, br,
             we1, be1, we2, be2, ws1, bs1, ws2, bs2,
             out_ref, xbf_s, g_s):
    s = pl.program_id(0)

    @pl.when(s == 0)
    def _():
        _prologue(x_ref[...], (n1w, n1b, wv, wo, n2w, n2b, wr, br),
                  out_ref, xbf_s, g_s)

    @pl.when((s >= 1) & (s <= E_STEPS))
    def _():
        _expert_step(s - 1, we1, be1, we2, be2, out_ref, xbf_s, g_s)

    @pl.when(s > E_STEPS)
    def _():
        _shared_step(s - (E_STEPS + 1), ws1, bs1, ws2, bs2,
                     out_ref, xbf_s, False)


def _layer1(x, lp):
    return pl.pallas_call(
        _l1_body,
        grid=(PHASE,),
        in_specs=[pl.BlockSpec((S_LEN, D_MODEL), lambda s: (0, 0))]
        + _layer_specs(0),
        out_specs=pl.BlockSpec((S_LEN, D_MODEL), lambda s: (0, 0)),
        out_shape=jax.ShapeDtypeStruct((S_LEN, D_MODEL), jnp.float32),
        scratch_shapes=[
            pltpu.VMEM((S_LEN, D_MODEL), jnp.bfloat16),
            pltpu.VMEM((S_LEN, N_EXP), jnp.float32),
        ],
    )(x, *_layer_args(lp))


def _l2k4_body(x_ref,
               n1w, n1b, wv, wo, n2w, n2b, wr, br,
               we1, be1, we2, be2, ws1, bs1, ws2, bs2,
               wout_ref, bout_ref,
               out_ref, x_cur, xbf_s, g_s):
    s = pl.program_id(0)

    @pl.when(s == 0)
    def _():
        _prologue(x_ref[...], (n1w, n1b, wv, wo, n2w, n2b, wr, br),
                  x_cur, xbf_s, g_s)

    @pl.when((s >= 1) & (s <= E_STEPS))
    def _():
        _expert_step(s - 1, we1, be1, we2, be2, x_cur, xbf_s, g_s)

    @pl.when((s > E_STEPS) & (s < PHASE))
    def _():
        _shared_step(s - (E_STEPS + 1), ws1, bs1, ws2, bs2,
                     x_cur, xbf_s, True)

    @pl.when(s >= PHASE)
    def _():
        out_ref[...] = _dot_t16(xbf_s[...], wout_ref[...]) + bout_ref[...]


def _l2k4(x, lp, wout, bout):
    def _v_idx(s):
        return jnp.clip(s - PHASE, 0, N_VBLK - 1)

    return pl.pallas_call(
        _l2k4_body,
        grid=(PHASE + N_VBLK,),
        in_specs=(
            [pl.BlockSpec((S_LEN, D_MODEL), lambda s: (0, 0))]
            + _layer_specs(0)
            + [
                pl.BlockSpec((VBLK, D_MODEL), lambda s: (_v_idx(s), 0)),
                pl.BlockSpec((1, VBLK), lambda s: (0, _v_idx(s))),
            ]
        ),
        out_specs=pl.BlockSpec((S_LEN, VBLK), lambda s: (0, _v_idx(s))),
        out_shape=jax.ShapeDtypeStruct((S_LEN, N_VOCAB), jnp.float32),
        scratch_shapes=[
            pltpu.VMEM((S_LEN, D_MODEL), jnp.float32),
            pltpu.VMEM((S_LEN, D_MODEL), jnp.bfloat16),
            pltpu.VMEM((S_LEN, N_EXP), jnp.float32),
        ],
    )(x, *_layer_args(lp), wout, bout.reshape(1, N_VOCAB))


def _tc_forward(x, params):
    lp_a, lp_b = params['layers']
    x1 = _layer1(x, lp_a)
    logits = _l2k4(x1, lp_b, params['Wout'], params['bout'])
    return logits.reshape(1, S_LEN, N_VOCAB)


def kernel(input_ids, params):
    ids = input_ids.reshape(-1).astype(jnp.int32)
    x = _embed_gather(params['embedding'], ids)
    return _tc_forward(x, params)


# speedup vs baseline: 1.3697x; 1.3697x over previous
import functools

import jax
import jax.numpy as jnp
from jax import lax
from jax.experimental import pallas as pl
from jax.experimental.pallas import tpu as pltpu
from jax.experimental.pallas import tpu_sc as plsc

S_LEN = 2048
D_MODEL = 1024
KV_R = 64
N_HD = 16
N_EXP = 16
M_INT = 512
S_INT = 2048
N_VOCAB = 32000
EPS = 1e-5
N_SBLK = S_INT // M_INT

SC_NC = 2
SC_NS = 16
SC_NW = SC_NC * SC_NS
BPW = S_LEN // SC_NW


def _ln_rows(x, w, b):
    mu = jnp.mean(x, axis=-1, keepdims=True)
    var = jnp.mean((x - mu) ** 2, axis=-1, keepdims=True)
    return (x - mu) / jnp.sqrt(var + EPS) * w + b


def _silu(x):
    return x * jax.nn.sigmoid(x)


def _dot_t(a, b):
    return lax.dot_general(a, b, (((1,), (1,)), ((), ())),
                           preferred_element_type=jnp.float32)


def _dot_t16(a, b):
    return lax.dot_general(a.astype(jnp.bfloat16), b.astype(jnp.bfloat16),
                           (((1,), (1,)), ((), ())),
                           preferred_element_type=jnp.float32)


def _embed_body(table_hbm, idx_hbm, out_hbm, idx_v, rows_v, sem):
    wid = lax.axis_index("s") * SC_NC + lax.axis_index("c")
    base = wid * BPW
    pltpu.sync_copy(idx_hbm.at[pl.ds(base, BPW)], idx_v)
    pltpu.async_copy(table_hbm.at[idx_v], rows_v, sem).wait()
    pltpu.sync_copy(rows_v, out_hbm.at[pl.ds(base, BPW)])


def _embed_gather(table, ids):
    mesh = plsc.VectorSubcoreMesh(core_axis_name="c", subcore_axis_name="s")
    k = functools.partial(
        pl.kernel,
        mesh=mesh,
        out_type=jax.ShapeDtypeStruct((S_LEN, D_MODEL), jnp.float32),
        scratch_types=[
            pltpu.VMEM((BPW,), jnp.int32),
            pltpu.VMEM((BPW, D_MODEL), jnp.float32),
            pltpu.SemaphoreType.DMA,
        ],
    )(_embed_body)
    return k(table, ids)


def _layer_body(x_ref, n1w_ref, n1b_ref, wv_ref, wo_ref, n2w_ref, n2b_ref,
                wr_ref, br_ref, we1_ref, be1_ref, we2_ref, be2_ref,
                ws1_ref, bs1_ref, ws2_ref, bs2_ref,
                out_ref, xln2_s, g_s):
    s = pl.program_id(0)

    @pl.when(s == 0)
    def _():
        x = x_ref[...]
        xln = _ln_rows(x, n1w_ref[...], n1b_ref[...])
        v = _dot_t(xln, wv_ref[...])
        vt = jnp.concatenate([v] * N_HD, axis=1)
        y = x + _dot_t(vt, wo_ref[...])
        out_ref[...] = y
        xln2 = _ln_rows(y, n2w_ref[...], n2b_ref[...])
        xln2_s[...] = xln2.astype(jnp.bfloat16)
        logits = _dot_t(xln2, wr_ref[...]) + br_ref[...]
        m = jnp.max(logits, axis=-1, keepdims=True)
        ex = jnp.exp(logits - m)
        w = ex / jnp.sum(ex, axis=-1, keepdims=True)
        m1 = jnp.max(w, axis=-1, keepdims=True)
        w2 = jnp.where(w >= m1, -1.0, w)
        m2 = jnp.max(w2, axis=-1, keepdims=True)
        sel = w >= m2
        g_s[...] = jnp.where(sel, w, 0.0) / (m1 + m2)

    @pl.when((s >= 1) & (s <= N_EXP))
    def _():
        e = s - 1
        xb = xln2_s[...]
        h = _silu(_dot_t16(xb, we1_ref[0]) + be1_ref[0])
        o = _dot_t16(h, we2_ref[0]) + be2_ref[0]
        lane = lax.broadcasted_iota(jnp.int32, (S_LEN, N_EXP), 1)
        ge = jnp.sum(jnp.where(lane == e, g_s[...], 0.0), axis=1,
                     keepdims=True)
        out_ref[...] = out_ref[...] + ge * o

    @pl.when(s > N_EXP)
    def _():
        c = s - (N_EXP + 1)
        xb = xln2_s[...]
        h = _silu(_dot_t16(xb, ws1_ref[...]) + bs1_ref[...])
        part = _dot_t16(h, ws2_ref[...])
        first = jnp.where(c == 0, 1.0, 0.0)
        out_ref[...] = out_ref[...] + part + first * bs2_ref[...]


def _layer(x, lp):
    def _e_idx(s):
        return jnp.clip(s - 1, 0, N_EXP - 1)

    def _c_idx(s):
        return jnp.clip(s - (N_EXP + 1), 0, N_SBLK - 1)

    return pl.pallas_call(
        _layer_body,
        grid=(1 + N_EXP + N_SBLK,),
        in_specs=[
            pl.BlockSpec((S_LEN, D_MODEL), lambda s: (0, 0)),
            pl.BlockSpec((1, D_MODEL), lambda s: (0, 0)),
            pl.BlockSpec((1, D_MODEL), lambda s: (0, 0)),
            pl.BlockSpec((KV_R, D_MODEL), lambda s: (0, 0)),
            pl.BlockSpec((D_MODEL, D_MODEL), lambda s: (0, 0)),
            pl.BlockSpec((1, D_MODEL), lambda s: (0, 0)),
            pl.BlockSpec((1, D_MODEL), lambda s: (0, 0)),
            pl.BlockSpec((N_EXP, D_MODEL), lambda s: (0, 0)),
            pl.BlockSpec((1, N_EXP), lambda s: (0, 0)),
            pl.BlockSpec((1, M_INT, D_MODEL),
                         lambda s: (_e_idx(s), 0, 0)),
            pl.BlockSpec((1, 1, M_INT), lambda s: (_e_idx(s), 0, 0)),
            pl.BlockSpec((1, D_MODEL, M_INT),
                         lambda s: (_e_idx(s), 0, 0)),
            pl.BlockSpec((1, 1, D_MODEL),
                         lambda s: (_e_idx(s), 0, 0)),
            pl.BlockSpec((M_INT, D_MODEL), lambda s: (_c_idx(s), 0)),
            pl.BlockSpec((1, M_INT), lambda s: (0, _c_idx(s))),
            pl.BlockSpec((D_MODEL, M_INT), lambda s: (0, _c_idx(s))),
            pl.BlockSpec((1, D_MODEL), lambda s: (0, 0)),
        ],
        out_specs=pl.BlockSpec((S_LEN, D_MODEL), lambda s: (0, 0)),
        out_shape=jax.ShapeDtypeStruct((S_LEN, D_MODEL), jnp.float32),
        scratch_shapes=[
            pltpu.VMEM((S_LEN, D_MODEL), jnp.bfloat16),
            pltpu.VMEM((S_LEN, N_EXP), jnp.float32),
        ],
    )(x,
      lp['n1w'].reshape(1, D_MODEL), lp['n1b'].reshape(1, D_MODEL),
      lp['Wv'], lp['Wo'],
      lp['n2w'].reshape(1, D_MODEL), lp['n2b'].reshape(1, D_MODEL),
      lp['Wr'], lp['br'].reshape(1, N_EXP),
      lp['We1'], lp['be1'].reshape(N_EXP, 1, M_INT),
      lp['We2'], lp['be2'].reshape(N_EXP, 1, D_MODEL),
      lp['Ws1'][0], lp['bs1'], lp['Ws2'][0], lp['bs2'])


def _k4_body(x_ref, wout_ref, bout_ref, out_ref):
    out_ref[...] = _dot_t16(x_ref[...], wout_ref[...]) + bout_ref[...]


def _k4(x, wout, bout):
    vblk = 1280
    nblk = N_VOCAB // vblk
    return pl.pallas_call(
        _k4_body,
        grid=(nblk,),
        in_specs=[
            pl.BlockSpec((S_LEN, D_MODEL), lambda j: (0, 0)),
            pl.BlockSpec((vblk, D_MODEL), lambda j: (j, 0)),
            pl.BlockSpec((1, vblk), lambda j: (0, j)),
        ],
        out_specs=pl.BlockSpec((S_LEN, vblk), lambda j: (0, j)),
        out_shape=jax.ShapeDtypeStruct((S_LEN, N_VOCAB), jnp.float32),
    )(x, wout, bout.reshape(1, N_VOCAB))


def _tc_forward(x, params):
    for lp in params['layers']:
        x = _layer(x, lp)
    logits = _k4(x.astype(jnp.bfloat16), params['Wout'], params['bout'])
    return logits.reshape(1, S_LEN, N_VOCAB)


def kernel(input_ids, params):
    ids = input_ids.reshape(-1).astype(jnp.int32)
    x = _embed_gather(params['embedding'], ids)
    return _tc_forward(x, params)
